# baseline (speedup 1.0000x reference)
import functools

import jax
import jax.numpy as jnp
from jax import lax
from jax.experimental import pallas as pl
from jax.experimental.pallas import tpu as pltpu
from jax.experimental.pallas import tpu_sc as plsc

NC = 2
NS = 16
L = 16
NW = NC * NS

N_RAW = 10000
E_RAW = 320000
D = 128

N_PAD = 10240
E_TOT = E_RAW + N_RAW
CH = 64
G_PER_TILE = -(-E_TOT // (NW * CH))
G_PAD = 176
E_PAD = G_PAD * NW * CH
ROWS_PER_TILE = N_PAD // NS
SLAB = 48
PHASES = ((0, 48, 48), (48, 48, 48), (96, 48, 48), (144, 32, 18))
G0, G1 = 200, 124
GP_PAD = 200
PSLAB = 40
K0S = (40, 40, 40, 40, 40)
K1S = (28, 24, 24, 24, 24)

def _z16():
    return jnp.zeros((L,), jnp.float32)


def _mesh():
    return plsc.VectorSubcoreMesh(
        core_axis_name="c", subcore_axis_name="s", num_cores=NC, num_subcores=NS
    )


DW = 128


def _deg_body(dst3_hbm, deg_hbm, dslab_v, val_v, sem0, sem1, deg_sh):
    c = lax.axis_index("c")
    s = lax.axis_index("s")
    wid = c * NS + s

    @pl.loop(0, CH)
    def _zf(r):
        for k in range(DW // L):
            val_v[r, pl.ds(k * L, L)] = _z16()

    @pl.loop(0, ROWS_PER_TILE // CH)
    def _zs(j):
        pltpu.sync_copy(val_v, deg_sh.at[pl.ds(s * ROWS_PER_TILE + j * CH, CH)])

    @pl.loop(0, CH)
    def _of(r):
        for k in range(DW // L):
            val_v[r, pl.ds(k * L, L)] = _z16() + 1.0

    plsc.subcore_barrier()

    def scat(g, sem):
        pltpu.async_copy(val_v, deg_sh.at[dslab_v.at[g]], sem, add=True)

    def wait_s(sem):
        pltpu.make_async_copy(val_v, deg_sh.at[dslab_v.at[0]], sem).wait()

    def sweep(k):
        scat(0, sem0)
        scat(1, sem1)

        @pl.loop(1, k // 2)
        def _edges(m):
            wait_s(sem0)
            scat(2 * m, sem0)
            wait_s(sem1)
            scat(2 * m + 1, sem1)

        if k % 2 == 1:
            wait_s(sem0)
            scat(k - 1, sem0)
        wait_s(sem0)
        wait_s(sem1)

    for off, rows, k in PHASES:
        pltpu.sync_copy(dst3_hbm.at[wid, pl.ds(off, rows)],
                        dslab_v.at[pl.ds(0, rows)])
        sweep(k)

    plsc.subcore_barrier()

    @pl.loop(0, ROWS_PER_TILE // CH)
    def _out(j):
        r0 = s * ROWS_PER_TILE + j * CH
        pltpu.sync_copy(deg_sh.at[pl.ds(r0, CH)], val_v)
        pltpu.sync_copy(val_v, deg_hbm.at[c, pl.ds(r0, CH)])


def _deg(dst3):
    f = functools.partial(
        pl.kernel,
        out_type=jax.ShapeDtypeStruct((NC, N_PAD, DW), jnp.float32),
        mesh=_mesh(),
        scratch_types=[
            pltpu.VMEM((SLAB, CH), jnp.int32),
            pltpu.VMEM((CH, DW), jnp.float32),
            pltpu.SemaphoreType.DMA,
            pltpu.SemaphoreType.DMA,
            pltpu.VMEM_SHARED((N_PAD, DW), jnp.float32),
        ],
    )(_deg_body)
    return f(dst3)


def _prop_body(hp_hbm, src3_hbm, dst3_hbm, out_hbm, sslab_v, dslab_v,
               rows0_v, rows1_v, rows2_v, rows3_v,
               gsem0, gsem1, gsem2, gsem3, ssem0, ssem1, ssem2, ssem3, acc_sh):
    c = lax.axis_index("c")
    s = lax.axis_index("s")
    wid = c * NS + s

    @pl.loop(0, CH)
    def _zr(r):
        for k in range(D // L):
            rows0_v[r, pl.ds(k * L, L)] = _z16()

    @pl.loop(0, ROWS_PER_TILE // CH)
    def _zs(j):
        pltpu.sync_copy(rows0_v, acc_sh.at[pl.ds(s * ROWS_PER_TILE + j * CH, CH)])

    plsc.subcore_barrier()

    def gath(g, buf, sem):
        pltpu.async_copy(hp_hbm.at[sslab_v.at[g]], buf, sem)

    def wait_g(buf, sem):
        pltpu.make_async_copy(hp_hbm.at[sslab_v.at[0]], buf, sem).wait()

    def scat(g, buf, sem):
        pltpu.async_copy(buf, acc_sh.at[dslab_v.at[g]], sem, add=True)

    def wait_s(buf, sem):
        pltpu.make_async_copy(buf, acc_sh.at[dslab_v.at[0]], sem).wait()

    bufs = [rows0_v, rows1_v, rows2_v, rows3_v]
    gsems = [gsem0, gsem1, gsem2, gsem3]
    ssems = [ssem0, ssem1, ssem2, ssem3]

    def sweep(k):
        for i in range(4):
            gath(i, bufs[i], gsems[i])
        for i in range(4):
            wait_g(bufs[i], gsems[i])
            scat(i, bufs[i], ssems[i])

        @pl.loop(1, k // 4)
        def _quads(m):
            for i in range(4):
                wait_s(bufs[i], ssems[i])
                gath(4 * m + i, bufs[i], gsems[i])
            for i in range(4):
                wait_g(bufs[i], gsems[i])
                scat(4 * m + i, bufs[i], ssems[i])

        for i in range(4):
            wait_s(bufs[i], ssems[i])

    for p in range(len(K0S)):
        pltpu.sync_copy(src3_hbm.at[wid, pl.ds(p * PSLAB, PSLAB)],
                        sslab_v.at[pl.ds(0, PSLAB)])
        pltpu.sync_copy(dst3_hbm.at[wid, pl.ds(p * PSLAB, PSLAB)],
                        dslab_v.at[pl.ds(0, PSLAB)])
        k = jnp.where(c == 0, K0S[p], K1S[p])
        sweep(k)

    plsc.subcore_barrier()

    @pl.loop(0, ROWS_PER_TILE // CH)
    def _out(j):
        r0 = s * ROWS_PER_TILE + j * CH
        pltpu.sync_copy(acc_sh.at[pl.ds(r0, CH)], rows0_v)
        pltpu.sync_copy(rows0_v, out_hbm.at[c, pl.ds(r0, CH)])


def _prop(hp, src3, dst3):
    f = functools.partial(
        pl.kernel,
        out_type=jax.ShapeDtypeStruct((NC, N_PAD, D), jnp.float32),
        mesh=_mesh(),
        scratch_types=[
            pltpu.VMEM((SLAB, CH), jnp.int32),
            pltpu.VMEM((SLAB, CH), jnp.int32),
            pltpu.VMEM((CH, D), jnp.float32),
            pltpu.VMEM((CH, D), jnp.float32),
            pltpu.VMEM((CH, D), jnp.float32),
            pltpu.VMEM((CH, D), jnp.float32),
            pltpu.SemaphoreType.DMA,
            pltpu.SemaphoreType.DMA,
            pltpu.SemaphoreType.DMA,
            pltpu.SemaphoreType.DMA,
            pltpu.SemaphoreType.DMA,
            pltpu.SemaphoreType.DMA,
            pltpu.SemaphoreType.DMA,
            pltpu.SemaphoreType.DMA,
            pltpu.VMEM_SHARED((N_PAD, D), jnp.float32),
        ],
    )(_prop_body)
    return f(hp, src3, dst3)


def _dis(d0, d1):
    deg = d0 + d1
    return jnp.where(deg > 0, lax.rsqrt(jnp.maximum(deg, 1e-12)), 0.0)


def _mm_in_body(x_ref, w_ref, d0_ref, d1_ref, o_ref):
    h = jnp.dot(x_ref[...], w_ref[...], preferred_element_type=jnp.float32)
    o_ref[...] = h * _dis(d0_ref[...], d1_ref[...])


def _mm_mid_body(a0_ref, a1_ref, d0_ref, d1_ref, b_ref, w_ref, o_ref):
    dis = _dis(d0_ref[...], d1_ref[...])
    pre = jnp.maximum(dis * (a0_ref[...] + a1_ref[...]) + b_ref[...], 0.0)
    h = jnp.dot(pre, w_ref[...], preferred_element_type=jnp.float32)
    o_ref[...] = h * dis


def _mm_fin_body(a0_ref, a1_ref, d0_ref, d1_ref, b_ref, w_ref, bo_ref, o_ref):
    dis = _dis(d0_ref[...], d1_ref[...])
    pre = jnp.maximum(dis * (a0_ref[...] + a1_ref[...]) + b_ref[...], 0.0)
    h = jnp.dot(pre, w_ref[...], preferred_element_type=jnp.float32)
    o_ref[...] = h + bo_ref[...]


_BM = 2048


def _row_spec():
    return pl.BlockSpec((_BM, D), lambda i: (i, 0))


def _d_spec():
    return pl.BlockSpec((_BM, 1), lambda i: (i, 0))


def _w_spec():
    return pl.BlockSpec((D, D), lambda i: (0, 0))


def _b_spec():
    return pl.BlockSpec((1, D), lambda i: (0, 0))


def _out_sds():
    return jax.ShapeDtypeStruct((N_PAD, D), jnp.float32)


def _mm_in(x, w, d0, d1):
    return pl.pallas_call(
        _mm_in_body,
        grid=(N_PAD // _BM,),
        in_specs=[_row_spec(), _w_spec(), _d_spec(), _d_spec()],
        out_specs=_row_spec(),
        out_shape=_out_sds(),
    )(x, w, d0, d1)


def _mm_mid(a0, a1, d0, d1, b, w):
    return pl.pallas_call(
        _mm_mid_body,
        grid=(N_PAD // _BM,),
        in_specs=[_row_spec(), _row_spec(), _d_spec(), _d_spec(), _b_spec(),
                  _w_spec()],
        out_specs=_row_spec(),
        out_shape=_out_sds(),
    )(a0, a1, d0, d1, b, w)


def _mm_fin(a0, a1, d0, d1, b, w, bo):
    return pl.pallas_call(
        _mm_fin_body,
        grid=(N_PAD // _BM,),
        in_specs=[_row_spec(), _row_spec(), _d_spec(), _d_spec(), _b_spec(),
                  _w_spec(), _b_spec()],
        out_specs=_row_spec(),
        out_shape=_out_sds(),
    )(a0, a1, d0, d1, b, w, bo)


def kernel(x, edge_index, W1, b1, W2, b2, Wfc, bfc):
    n = x.shape[0]
    loop_idx = jnp.arange(n, dtype=jnp.int32)
    pad_e = G_PER_TILE * NW * CH - E_RAW - n
    pad_idx = jnp.full((pad_e,), n, dtype=jnp.int32)
    src = jnp.concatenate([edge_index[0].astype(jnp.int32), loop_idx, pad_idx])
    dst = jnp.concatenate([edge_index[1].astype(jnp.int32), loop_idx, pad_idx])
    dstb = jnp.pad(dst.reshape(NW, G_PER_TILE, CH),
                   ((0, 0), (0, G_PAD - G_PER_TILE), (0, 0)),
                   constant_values=n)

    def asym(a):
        a0 = a[: NS * G0 * CH].reshape(NS, G0, CH)
        a1 = a[NS * G0 * CH:].reshape(NS, G1, CH)
        a0 = jnp.pad(a0, ((0, 0), (0, GP_PAD - G0), (0, 0)), constant_values=n)
        a1 = jnp.pad(a1, ((0, 0), (0, GP_PAD - G1), (0, 0)), constant_values=n)
        return jnp.concatenate([a0, a1], axis=0)

    srca = asym(src)
    dsta = asym(dst)

    x_pad = jnp.zeros((N_PAD, D), jnp.float32).at[:n].set(x)

    deg = _deg(dstb)
    d0 = deg[0, :, 0:1]
    d1 = deg[1, :, 0:1]

    b1r = b1[None, :]
    b2r = b2[None, :]
    bfr = bfc[None, :]

    h1 = _mm_in(x_pad, W1, d0, d1)
    a1 = _prop(h1, srca, dsta)
    h2 = _mm_mid(a1[0], a1[1], d0, d1, b1r, W2)
    a2 = _prop(h2, srca, dsta)
    y = _mm_fin(a2[0], a2[1], d0, d1, b2r, Wfc, bfr)
    return y[:n]


# speedup vs baseline: 4.2624x; 4.2624x over previous
import functools

import jax
import jax.numpy as jnp
from jax import lax
from jax.experimental import pallas as pl
from jax.experimental.pallas import tpu as pltpu
from jax.experimental.pallas import tpu_sc as plsc

NC = 2
NS = 16
L = 16
NW = NC * NS

N_RAW = 10000
E_RAW = 320000
D = 128

N_PAD = 10240
E_TOT = E_RAW + N_RAW
CH = 32
G_PER_TILE = -(-E_TOT // (NW * CH))
G_PAD = 352
E_PAD = G_PAD * NW * CH
ROWS_PER_TILE = N_PAD // NS
SLAB = 32
PHASES = tuple((32 * p, 32) for p in range(10)) + ((320, 8),)

def _z16():
    return jnp.zeros((L,), jnp.float32)


def _mesh():
    return plsc.VectorSubcoreMesh(
        core_axis_name="c", subcore_axis_name="s", num_cores=NC, num_subcores=NS
    )


DW = 128


def _deg_body(dst3_hbm, deg_hbm, dslab_v, val_v, sem0, sem1, deg_sh):
    c = lax.axis_index("c")
    s = lax.axis_index("s")
    wid = c * NS + s

    @pl.loop(0, CH)
    def _zf(r):
        for k in range(DW // L):
            val_v[r, pl.ds(k * L, L)] = _z16()

    @pl.loop(0, ROWS_PER_TILE // CH)
    def _zs(j):
        pltpu.sync_copy(val_v, deg_sh.at[pl.ds(s * ROWS_PER_TILE + j * CH, CH)])

    @pl.loop(0, CH)
    def _of(r):
        for k in range(DW // L):
            val_v[r, pl.ds(k * L, L)] = _z16() + 1.0

    plsc.subcore_barrier()

    def scat(g, sem):
        pltpu.async_copy(val_v, deg_sh.at[dslab_v.at[g]], sem, add=True)

    def wait_s(sem):
        pltpu.make_async_copy(val_v, deg_sh.at[dslab_v.at[0]], sem).wait()

    def sweep(k):
        scat(0, sem0)
        scat(1, sem1)

        @pl.loop(1, k // 2)
        def _edges(m):
            wait_s(sem0)
            scat(2 * m, sem0)
            wait_s(sem1)
            scat(2 * m + 1, sem1)

        if k % 2 == 1:
            wait_s(sem0)
            scat(k - 1, sem0)
        wait_s(sem0)
        wait_s(sem1)

    for off, k in PHASES:
        pltpu.sync_copy(dst3_hbm.at[wid, pl.ds(off, SLAB)],
                        dslab_v.at[pl.ds(0, SLAB)])
        sweep(k)

    plsc.subcore_barrier()

    @pl.loop(0, ROWS_PER_TILE // CH)
    def _out(j):
        r0 = s * ROWS_PER_TILE + j * CH
        pltpu.sync_copy(deg_sh.at[pl.ds(r0, CH)], val_v)
        pltpu.sync_copy(val_v, deg_hbm.at[c, pl.ds(r0, CH)])


def _deg(dst3):
    f = functools.partial(
        pl.kernel,
        out_type=jax.ShapeDtypeStruct((NC, N_PAD, DW), jnp.float32),
        mesh=_mesh(),
        scratch_types=[
            pltpu.VMEM((SLAB, CH), jnp.int32),
            pltpu.VMEM((CH, DW), jnp.float32),
            pltpu.SemaphoreType.DMA,
            pltpu.SemaphoreType.DMA,
            pltpu.VMEM_SHARED((N_PAD, DW), jnp.float32),
        ],
    )(_deg_body)
    return f(dst3)


def _prop_body(hp_hbm, src3_hbm, dst3_hbm, out_hbm, sslab_v, dslab_v,
               fb0_v, fb1_v, fb2_v, fb3_v, fb4_v, fb5_v, fb6_v, fb7_v,
               gsem0, gsem1, gsem2, gsem3, gsem4, gsem5, gsem6, gsem7,
               ssem0, ssem1, ssem2, ssem3, ssem4, ssem5, ssem6, ssem7,
               acc_sh):
    c = lax.axis_index("c")
    s = lax.axis_index("s")
    wid = c * NS + s

    @pl.loop(0, CH)
    def _zr(r):
        for k in range(D // L):
            fb0_v[r, pl.ds(k * L, L)] = _z16()

    @pl.loop(0, ROWS_PER_TILE // CH)
    def _zs(j):
        pltpu.sync_copy(fb0_v, acc_sh.at[pl.ds(s * ROWS_PER_TILE + j * CH, CH)])

    plsc.subcore_barrier()

    fbs = [fb0_v, fb1_v, fb2_v, fb3_v, fb4_v, fb5_v, fb6_v, fb7_v]
    gsems = [gsem0, gsem1, gsem2, gsem3, gsem4, gsem5, gsem6, gsem7]
    ssems = [ssem0, ssem1, ssem2, ssem3, ssem4, ssem5, ssem6, ssem7]

    def gath(g, i):
        pltpu.async_copy(hp_hbm.at[sslab_v.at[g]], fbs[i], gsems[i])

    def wait_g(i):
        pltpu.make_async_copy(hp_hbm.at[sslab_v.at[0]], fbs[i], gsems[i]).wait()

    def scat(g, i):
        pltpu.async_copy(fbs[i], acc_sh.at[dslab_v.at[g]], ssems[i], add=True)

    def wait_s(i):
        pltpu.make_async_copy(fbs[i], acc_sh.at[dslab_v.at[0]], ssems[i]).wait()

    def sweep(k):
        for i in range(8):
            gath(i, i)
        for i in range(8):
            wait_g(i)
            scat(i, i)

        @pl.loop(1, k // 8)
        def _octs(m):
            for i in range(8):
                wait_s(i)
                gath(8 * m + i, i)
            for i in range(8):
                wait_g(i)
                scat(8 * m + i, i)

        for i in range(8):
            wait_s(i)

    for off, k in PHASES:
        pltpu.sync_copy(src3_hbm.at[wid, pl.ds(off, SLAB)],
                        sslab_v.at[pl.ds(0, SLAB)])
        pltpu.sync_copy(dst3_hbm.at[wid, pl.ds(off, SLAB)],
                        dslab_v.at[pl.ds(0, SLAB)])
        sweep(k)

    plsc.subcore_barrier()

    @pl.loop(0, ROWS_PER_TILE // CH)
    def _out(j):
        r0 = s * ROWS_PER_TILE + j * CH
        pltpu.sync_copy(acc_sh.at[pl.ds(r0, CH)], fb0_v)
        pltpu.sync_copy(fb0_v, out_hbm.at[c, pl.ds(r0, CH)])


def _prop(hp, src3, dst3):
    f = functools.partial(
        pl.kernel,
        out_type=jax.ShapeDtypeStruct((NC, N_PAD, D), jnp.float32),
        mesh=_mesh(),
        scratch_types=[
            pltpu.VMEM((SLAB, CH), jnp.int32),
            pltpu.VMEM((SLAB, CH), jnp.int32),
        ] + [pltpu.VMEM((CH, D), jnp.float32)] * 8
          + [pltpu.SemaphoreType.DMA] * 16
          + [pltpu.VMEM_SHARED((N_PAD, D), jnp.float32)],
    )(_prop_body)
    return f(hp, src3, dst3)


def _dis(d0, d1):
    deg = d0 + d1
    return jnp.where(deg > 0, lax.rsqrt(jnp.maximum(deg, 1e-12)), 0.0)


def _mm_in_body(x_ref, w_ref, d0_ref, d1_ref, o_ref):
    h = jnp.dot(x_ref[...], w_ref[...], preferred_element_type=jnp.float32)
    o_ref[...] = h * _dis(d0_ref[...], d1_ref[...])


def _mm_mid_body(a0_ref, a1_ref, d0_ref, d1_ref, b_ref, w_ref, o_ref):
    dis = _dis(d0_ref[...], d1_ref[...])
    pre = jnp.maximum(dis * (a0_ref[...] + a1_ref[...]) + b_ref[...], 0.0)
    h = jnp.dot(pre, w_ref[...], preferred_element_type=jnp.float32)
    o_ref[...] = h * dis


def _mm_fin_body(a0_ref, a1_ref, d0_ref, d1_ref, b_ref, w_ref, bo_ref, o_ref):
    dis = _dis(d0_ref[...], d1_ref[...])
    pre = jnp.maximum(dis * (a0_ref[...] + a1_ref[...]) + b_ref[...], 0.0)
    h = jnp.dot(pre, w_ref[...], preferred_element_type=jnp.float32)
    o_ref[...] = h + bo_ref[...]


_BM = 2048


def _row_spec():
    return pl.BlockSpec((_BM, D), lambda i: (i, 0))


def _d_spec():
    return pl.BlockSpec((_BM, 1), lambda i: (i, 0))


def _w_spec():
    return pl.BlockSpec((D, D), lambda i: (0, 0))


def _b_spec():
    return pl.BlockSpec((1, D), lambda i: (0, 0))


def _out_sds():
    return jax.ShapeDtypeStruct((N_PAD, D), jnp.float32)


def _mm_in(x, w, d0, d1):
    return pl.pallas_call(
        _mm_in_body,
        grid=(N_PAD // _BM,),
        in_specs=[_row_spec(), _w_spec(), _d_spec(), _d_spec()],
        out_specs=_row_spec(),
        out_shape=_out_sds(),
    )(x, w, d0, d1)


def _mm_mid(a0, a1, d0, d1, b, w):
    return pl.pallas_call(
        _mm_mid_body,
        grid=(N_PAD // _BM,),
        in_specs=[_row_spec(), _row_spec(), _d_spec(), _d_spec(), _b_spec(),
                  _w_spec()],
        out_specs=_row_spec(),
        out_shape=_out_sds(),
    )(a0, a1, d0, d1, b, w)


def _mm_fin(a0, a1, d0, d1, b, w, bo):
    return pl.pallas_call(
        _mm_fin_body,
        grid=(N_PAD // _BM,),
        in_specs=[_row_spec(), _row_spec(), _d_spec(), _d_spec(), _b_spec(),
                  _w_spec(), _b_spec()],
        out_specs=_row_spec(),
        out_shape=_out_sds(),
    )(a0, a1, d0, d1, b, w, bo)


def kernel(x, edge_index, W1, b1, W2, b2, Wfc, bfc):
    n = x.shape[0]
    loop_idx = jnp.arange(n, dtype=jnp.int32)
    pad_e = G_PER_TILE * NW * CH - E_RAW - n
    pad_idx = jnp.full((pad_e,), n, dtype=jnp.int32)
    src = jnp.concatenate([edge_index[0].astype(jnp.int32), loop_idx, pad_idx])
    dst = jnp.concatenate([edge_index[1].astype(jnp.int32), loop_idx, pad_idx])
    src = jnp.pad(src.reshape(NW, G_PER_TILE, CH),
                  ((0, 0), (0, G_PAD - G_PER_TILE), (0, 0)),
                  constant_values=n)
    dst = jnp.pad(dst.reshape(NW, G_PER_TILE, CH),
                  ((0, 0), (0, G_PAD - G_PER_TILE), (0, 0)),
                  constant_values=n)

    x_pad = jnp.zeros((N_PAD, D), jnp.float32).at[:n].set(x)

    deg = _deg(dst)
    d0 = deg[0, :, 0:1]
    d1 = deg[1, :, 0:1]

    b1r = b1[None, :]
    b2r = b2[None, :]
    bfr = bfc[None, :]

    h1 = _mm_in(x_pad, W1, d0, d1)
    a1 = _prop(h1, src, dst)
    h2 = _mm_mid(a1[0], a1[1], d0, d1, b1r, W2)
    a2 = _prop(h2, src, dst)
    y = _mm_fin(a2[0], a2[1], d0, d1, b2r, Wfc, bfr)
    return y[:n]


# speedup vs baseline: 6.8801x; 1.6141x over previous
import functools

import jax
import jax.numpy as jnp
from jax import lax
from jax.experimental import pallas as pl
from jax.experimental.pallas import tpu as pltpu
from jax.experimental.pallas import tpu_sc as plsc

NC = 2
NS = 16
L = 16
NW = NC * NS

N_RAW = 10000
E_RAW = 320000
D = 128

N_PAD = 10240
E_TOT = E_RAW + N_RAW
CH = 64
G_PER_TILE = -(-E_TOT // (NW * CH))
G_PAD = 176
E_PAD = G_PAD * NW * CH
ROWS_PER_TILE = N_PAD // NS
SLAB = 48
PHASES = ((0, 48, 48), (48, 48, 48), (96, 48, 48), (144, 32, 18))

def _z16():
    return jnp.zeros((L,), jnp.float32)


def _mesh():
    return plsc.VectorSubcoreMesh(
        core_axis_name="c", subcore_axis_name="s", num_cores=NC, num_subcores=NS
    )


DW = 128


def _deg_body(dst3_hbm, deg_hbm, dslab_v, val_v, sem0, sem1, deg_sh):
    c = lax.axis_index("c")
    s = lax.axis_index("s")
    wid = c * NS + s

    @pl.loop(0, CH)
    def _zf(r):
        for k in range(DW // L):
            val_v[r, pl.ds(k * L, L)] = _z16()

    @pl.loop(0, ROWS_PER_TILE // CH)
    def _zs(j):
        pltpu.sync_copy(val_v, deg_sh.at[pl.ds(s * ROWS_PER_TILE + j * CH, CH)])

    @pl.loop(0, CH)
    def _of(r):
        for k in range(DW // L):
            val_v[r, pl.ds(k * L, L)] = _z16() + 1.0

    plsc.subcore_barrier()

    def scat(g, sem):
        pltpu.async_copy(val_v, deg_sh.at[dslab_v.at[g]], sem, add=True)

    def wait_s(sem):
        pltpu.make_async_copy(val_v, deg_sh.at[dslab_v.at[0]], sem).wait()

    def sweep(k):
        scat(0, sem0)
        scat(1, sem1)

        @pl.loop(1, k // 2)
        def _edges(m):
            wait_s(sem0)
            scat(2 * m, sem0)
            wait_s(sem1)
            scat(2 * m + 1, sem1)

        if k % 2 == 1:
            wait_s(sem0)
            scat(k - 1, sem0)
        wait_s(sem0)
        wait_s(sem1)

    for off, rows, k in PHASES:
        pltpu.sync_copy(dst3_hbm.at[wid, pl.ds(off, rows)],
                        dslab_v.at[pl.ds(0, rows)])
        sweep(k)

    plsc.subcore_barrier()

    @pl.loop(0, ROWS_PER_TILE // CH)
    def _out(j):
        r0 = s * ROWS_PER_TILE + j * CH
        pltpu.sync_copy(deg_sh.at[pl.ds(r0, CH)], val_v)
        pltpu.sync_copy(val_v, deg_hbm.at[c, pl.ds(r0, CH)])


def _deg(dst3):
    f = functools.partial(
        pl.kernel,
        out_type=jax.ShapeDtypeStruct((NC, N_PAD, DW), jnp.float32),
        mesh=_mesh(),
        scratch_types=[
            pltpu.VMEM((SLAB, CH), jnp.int32),
            pltpu.VMEM((CH, DW), jnp.float32),
            pltpu.SemaphoreType.DMA,
            pltpu.SemaphoreType.DMA,
            pltpu.VMEM_SHARED((N_PAD, DW), jnp.float32),
        ],
    )(_deg_body)
    return f(dst3)


def _prop_body(hp_hbm, src3_hbm, dst3_hbm, out_hbm, sslab_v, dslab_v,
               rows0_v, rows1_v, rows2_v, rows3_v,
               gsem0, gsem1, gsem2, gsem3, ssem0, ssem1, ssem2, ssem3, acc_sh):
    c = lax.axis_index("c")
    s = lax.axis_index("s")
    wid = c * NS + s

    @pl.loop(0, CH)
    def _zr(r):
        for k in range(D // L):
            rows0_v[r, pl.ds(k * L, L)] = _z16()

    @pl.loop(0, ROWS_PER_TILE // CH)
    def _zs(j):
        pltpu.sync_copy(rows0_v, acc_sh.at[pl.ds(s * ROWS_PER_TILE + j * CH, CH)])

    plsc.subcore_barrier()

    def gath(g, buf, sem):
        pltpu.async_copy(hp_hbm.at[sslab_v.at[g]], buf, sem)

    def wait_g(buf, sem):
        pltpu.make_async_copy(hp_hbm.at[sslab_v.at[0]], buf, sem).wait()

    def scat(g, buf, sem):
        pltpu.async_copy(buf, acc_sh.at[dslab_v.at[g]], sem, add=True)

    def wait_s(buf, sem):
        pltpu.make_async_copy(buf, acc_sh.at[dslab_v.at[0]], sem).wait()

    bufs = [rows0_v, rows1_v, rows2_v, rows3_v]
    gsems = [gsem0, gsem1, gsem2, gsem3]
    ssems = [ssem0, ssem1, ssem2, ssem3]

    def sweep(k):
        for i in range(4):
            gath(i, bufs[i], gsems[i])
        for i in range(4):
            wait_g(bufs[i], gsems[i])
            scat(i, bufs[i], ssems[i])

        @pl.loop(1, k // 4)
        def _quads(m):
            for i in range(4):
                wait_s(bufs[i], ssems[i])
                gath(4 * m + i, bufs[i], gsems[i])
            for i in range(4):
                wait_g(bufs[i], gsems[i])
                scat(4 * m + i, bufs[i], ssems[i])

        for i in range(k % 4):
            wait_s(bufs[i], ssems[i])
            gath((k // 4) * 4 + i, bufs[i], gsems[i])
            wait_g(bufs[i], gsems[i])
            scat((k // 4) * 4 + i, bufs[i], ssems[i])
        for i in range(4):
            wait_s(bufs[i], ssems[i])

    for off, rows, k in PHASES:
        pltpu.sync_copy(src3_hbm.at[wid, pl.ds(off, rows)],
                        sslab_v.at[pl.ds(0, rows)])
        pltpu.sync_copy(dst3_hbm.at[wid, pl.ds(off, rows)],
                        dslab_v.at[pl.ds(0, rows)])
        sweep(k)

    plsc.subcore_barrier()

    @pl.loop(0, ROWS_PER_TILE // CH)
    def _out(j):
        r0 = s * ROWS_PER_TILE + j * CH
        pltpu.sync_copy(acc_sh.at[pl.ds(r0, CH)], rows0_v)
        pltpu.sync_copy(rows0_v, out_hbm.at[c, pl.ds(r0, CH)])


def _prop(hp, src3, dst3):
    f = functools.partial(
        pl.kernel,
        out_type=jax.ShapeDtypeStruct((NC, N_PAD, D), jnp.float32),
        mesh=_mesh(),
        scratch_types=[
            pltpu.VMEM((SLAB, CH), jnp.int32),
            pltpu.VMEM((SLAB, CH), jnp.int32),
            pltpu.VMEM((CH, D), jnp.float32),
            pltpu.VMEM((CH, D), jnp.float32),
            pltpu.VMEM((CH, D), jnp.float32),
            pltpu.VMEM((CH, D), jnp.float32),
            pltpu.SemaphoreType.DMA,
            pltpu.SemaphoreType.DMA,
            pltpu.SemaphoreType.DMA,
            pltpu.SemaphoreType.DMA,
            pltpu.SemaphoreType.DMA,
            pltpu.SemaphoreType.DMA,
            pltpu.SemaphoreType.DMA,
            pltpu.SemaphoreType.DMA,
            pltpu.VMEM_SHARED((N_PAD, D), jnp.float32),
        ],
    )(_prop_body)
    return f(hp, src3, dst3)


def _dis(d0, d1):
    deg = d0 + d1
    return jnp.where(deg > 0, lax.rsqrt(jnp.maximum(deg, 1e-12)), 0.0)


def _mm_in_body(x_ref, w_ref, d0_ref, d1_ref, o_ref):
    h = jnp.dot(x_ref[...], w_ref[...], preferred_element_type=jnp.float32)
    o_ref[...] = h * _dis(d0_ref[...], d1_ref[...])


def _mm_mid_body(a0_ref, a1_ref, d0_ref, d1_ref, b_ref, w_ref, o_ref):
    dis = _dis(d0_ref[...], d1_ref[...])
    pre = jnp.maximum(dis * (a0_ref[...] + a1_ref[...]) + b_ref[...], 0.0)
    h = jnp.dot(pre, w_ref[...], preferred_element_type=jnp.float32)
    o_ref[...] = h * dis


def _mm_fin_body(a0_ref, a1_ref, d0_ref, d1_ref, b_ref, w_ref, bo_ref, o_ref):
    dis = _dis(d0_ref[...], d1_ref[...])
    pre = jnp.maximum(dis * (a0_ref[...] + a1_ref[...]) + b_ref[...], 0.0)
    h = jnp.dot(pre, w_ref[...], preferred_element_type=jnp.float32)
    o_ref[...] = h + bo_ref[...]


_BM = 2048


def _row_spec():
    return pl.BlockSpec((_BM, D), lambda i: (i, 0))


def _d_spec():
    return pl.BlockSpec((_BM, 1), lambda i: (i, 0))


def _w_spec():
    return pl.BlockSpec((D, D), lambda i: (0, 0))


def _b_spec():
    return pl.BlockSpec((1, D), lambda i: (0, 0))


def _out_sds():
    return jax.ShapeDtypeStruct((N_PAD, D), jnp.float32)


def _mm_in(x, w, d0, d1):
    return pl.pallas_call(
        _mm_in_body,
        grid=(N_PAD // _BM,),
        in_specs=[_row_spec(), _w_spec(), _d_spec(), _d_spec()],
        out_specs=_row_spec(),
        out_shape=_out_sds(),
    )(x, w, d0, d1)


def _mm_mid(a0, a1, d0, d1, b, w):
    return pl.pallas_call(
        _mm_mid_body,
        grid=(N_PAD // _BM,),
        in_specs=[_row_spec(), _row_spec(), _d_spec(), _d_spec(), _b_spec(),
                  _w_spec()],
        out_specs=_row_spec(),
        out_shape=_out_sds(),
    )(a0, a1, d0, d1, b, w)


def _mm_fin(a0, a1, d0, d1, b, w, bo):
    return pl.pallas_call(
        _mm_fin_body,
        grid=(N_PAD // _BM,),
        in_specs=[_row_spec(), _row_spec(), _d_spec(), _d_spec(), _b_spec(),
                  _w_spec(), _b_spec()],
        out_specs=_row_spec(),
        out_shape=_out_sds(),
    )(a0, a1, d0, d1, b, w, bo)


def kernel(x, edge_index, W1, b1, W2, b2, Wfc, bfc):
    n = x.shape[0]
    loop_idx = jnp.arange(n, dtype=jnp.int32)
    pad_e = G_PER_TILE * NW * CH - E_RAW - n
    pad_idx = jnp.full((pad_e,), n, dtype=jnp.int32)
    src = jnp.concatenate([edge_index[0].astype(jnp.int32), loop_idx, pad_idx])
    dst = jnp.concatenate([edge_index[1].astype(jnp.int32), loop_idx, pad_idx])
    src = jnp.pad(src.reshape(NW, G_PER_TILE, CH),
                  ((0, 0), (0, G_PAD - G_PER_TILE), (0, 0)),
                  constant_values=n)
    dst = jnp.pad(dst.reshape(NW, G_PER_TILE, CH),
                  ((0, 0), (0, G_PAD - G_PER_TILE), (0, 0)),
                  constant_values=n)

    x_pad = jnp.zeros((N_PAD, D), jnp.float32).at[:n].set(x)

    deg = _deg(dst)
    d0 = deg[0, :, 0:1]
    d1 = deg[1, :, 0:1]

    b1r = b1[None, :]
    b2r = b2[None, :]
    bfr = bfc[None, :]

    h1 = _mm_in(x_pad, W1, d0, d1)
    a1 = _prop(h1, src, dst)
    h2 = _mm_mid(a1[0], a1[1], d0, d1, b1r, W2)
    a2 = _prop(h2, src, dst)
    y = _mm_fin(a2[0], a2[1], d0, d1, b2r, Wfc, bfr)
    return y[:n]
